# baseline (speedup 1.0000x reference)
import functools

import jax
import jax.numpy as jnp
from jax.experimental import pallas as pl
from jax.experimental.pallas import tpu as pltpu
from jax.experimental.pallas import tpu_sc as plsc


def _sc_gather_rows(table_eff, seg_pad, n_rows_pad, emb_dim, n_workers, nc):
    b_per_w = n_rows_pad // n_workers

    @functools.partial(
        pl.kernel,
        mesh=plsc.VectorSubcoreMesh(core_axis_name="c", subcore_axis_name="s"),
        out_type=jax.ShapeDtypeStruct((n_rows_pad, emb_dim), jnp.float32),
        scratch_types=[
            pltpu.VMEM((b_per_w,), jnp.int32),
            pltpu.VMEM((b_per_w, emb_dim), jnp.float32),
            pltpu.SemaphoreType.DMA,
        ],
    )
    def sc_gather(table_hbm, idx_hbm, out_hbm, idx_v, rows_v, sem):
        wid = jax.lax.axis_index("s") * nc + jax.lax.axis_index("c")
        base = wid * b_per_w
        pltpu.sync_copy(idx_hbm.at[pl.ds(base, b_per_w)], idx_v)
        pltpu.async_copy(table_hbm.at[idx_v], rows_v, sem).wait()
        pltpu.sync_copy(rows_v, out_hbm.at[pl.ds(base, b_per_w)])

    return sc_gather(table_eff, seg_pad)


def kernel(x, W, b, table, seg):
    B, P, DIN = x.shape
    EMB = W.shape[1]

    info = plsc.get_sparse_core_info()
    nc, ns = info.num_cores, info.num_subcores
    nw = nc * ns
    align = 8 * nw
    p_pad = ((P + align - 1) // align) * align

    emb_lanes = 128
    table_eff = jnp.zeros((table.shape[0], emb_lanes), jnp.float32)
    table_eff = table_eff.at[:, :EMB].set(table + b[None, :])
    seg_pad = jnp.concatenate(
        [seg.astype(jnp.int32), jnp.zeros((p_pad - P,), jnp.int32)]
    )

    emb_pad = _sc_gather_rows(table_eff, seg_pad, p_pad, emb_lanes, nw, nc)

    BB = 8

    def tc_body(x_ref, w_ref, emb_ref, out_ref):
        x2 = x_ref[...].reshape(BB * P, DIN)
        y = jnp.dot(x2, w_ref[...], preferred_element_type=jnp.float32)
        out_ref[...] = y.reshape(BB, P, EMB) + emb_ref[:, :EMB][None, :, :]

    out = pl.pallas_call(
        tc_body,
        grid=(B // BB,),
        in_specs=[
            pl.BlockSpec((BB, P, DIN), lambda i: (i, 0, 0)),
            pl.BlockSpec((DIN, EMB), lambda i: (0, 0)),
            pl.BlockSpec((P, emb_lanes), lambda i: (0, 0)),
        ],
        out_specs=pl.BlockSpec((BB, P, EMB), lambda i: (i, 0, 0)),
        out_shape=jax.ShapeDtypeStruct((B, P, EMB), jnp.float32),
    )(x, W, emb_pad)
    return out


# speedup vs baseline: 4.7639x; 4.7639x over previous
import functools

import jax
import jax.numpy as jnp
from jax.experimental import pallas as pl
from jax.experimental.pallas import tpu as pltpu
from jax.experimental.pallas import tpu_sc as plsc


def _sc_gather_rows(table_eff, seg_pad, n_rows_pad, row_lanes, n_workers, nc):
    b_per_w = n_rows_pad // n_workers

    @functools.partial(
        pl.kernel,
        mesh=plsc.VectorSubcoreMesh(core_axis_name="c", subcore_axis_name="s"),
        out_type=jax.ShapeDtypeStruct((n_rows_pad, row_lanes), jnp.float32),
        scratch_types=[
            pltpu.VMEM((b_per_w,), jnp.int32),
            pltpu.VMEM((b_per_w, row_lanes), jnp.float32),
            pltpu.SemaphoreType.DMA,
        ],
    )
    def sc_gather(table_hbm, idx_hbm, out_hbm, idx_v, rows_v, sem):
        wid = jax.lax.axis_index("s") * nc + jax.lax.axis_index("c")
        base = wid * b_per_w
        pltpu.sync_copy(idx_hbm.at[pl.ds(base, b_per_w)], idx_v)
        pltpu.async_copy(table_hbm.at[idx_v], rows_v, sem).wait()
        pltpu.sync_copy(rows_v, out_hbm.at[pl.ds(base, b_per_w)])

    return sc_gather(table_eff, seg_pad)


def kernel(x, W, b, table, seg):
    B, P, DIN = x.shape
    EMB = W.shape[1]

    info = plsc.get_sparse_core_info()
    nc, ns = info.num_cores, info.num_subcores
    nw = nc * ns
    align = 8 * nw
    p_pad = ((P + align - 1) // align) * align

    emb_lanes = 128
    table_eff = jnp.zeros((table.shape[0], emb_lanes), jnp.float32)
    table_eff = table_eff.at[:, :EMB].set(table + b[None, :])
    seg_pad = jnp.concatenate(
        [seg.astype(jnp.int32), jnp.zeros((p_pad - P,), jnp.int32)]
    )

    emb_pad = _sc_gather_rows(table_eff, seg_pad, p_pad, emb_lanes, nw, nc)
    emb3 = emb_pad[:P, :EMB][:, :, None]

    xt = jnp.transpose(x, (1, 2, 0))

    PP = 8
    wd = jnp.kron(jnp.eye(PP, dtype=W.dtype), W.T)

    def tc_body(xt_ref, wd_ref, emb_ref, out_ref):
        rhs = xt_ref[...].reshape(PP * DIN, B)
        y = jnp.dot(wd_ref[...], rhs, preferred_element_type=jnp.float32)
        out_ref[...] = y.reshape(PP, EMB, B) + emb_ref[...]

    out_t = pl.pallas_call(
        tc_body,
        grid=(P // PP,),
        in_specs=[
            pl.BlockSpec((PP, DIN, B), lambda i: (i, 0, 0)),
            pl.BlockSpec((PP * EMB, PP * DIN), lambda i: (0, 0)),
            pl.BlockSpec((PP, EMB, 1), lambda i: (i, 0, 0)),
        ],
        out_specs=pl.BlockSpec((PP, EMB, B), lambda i: (i, 0, 0)),
        out_shape=jax.ShapeDtypeStruct((P, EMB, B), jnp.float32),
    )(xt, wd, emb3)

    return jnp.transpose(out_t, (2, 0, 1))


# speedup vs baseline: 6.3961x; 1.3426x over previous
import functools

import jax
import jax.numpy as jnp
from jax.experimental import pallas as pl
from jax.experimental.pallas import tpu as pltpu
from jax.experimental.pallas import tpu_sc as plsc


def _sc_gather_rows(table_eff, seg_pad, n_rows_pad, row_lanes, n_workers, nc):
    b_per_w = n_rows_pad // n_workers

    @functools.partial(
        pl.kernel,
        mesh=plsc.VectorSubcoreMesh(core_axis_name="c", subcore_axis_name="s"),
        out_type=jax.ShapeDtypeStruct((n_rows_pad, row_lanes), jnp.float32),
        scratch_types=[
            pltpu.VMEM((b_per_w,), jnp.int32),
            pltpu.VMEM((b_per_w, row_lanes), jnp.float32),
            pltpu.SemaphoreType.DMA,
        ],
    )
    def sc_gather(table_hbm, idx_hbm, out_hbm, idx_v, rows_v, sem):
        wid = jax.lax.axis_index("s") * nc + jax.lax.axis_index("c")
        base = wid * b_per_w
        pltpu.sync_copy(idx_hbm.at[pl.ds(base, b_per_w)], idx_v)
        pltpu.async_copy(table_hbm.at[idx_v], rows_v, sem).wait()
        pltpu.sync_copy(rows_v, out_hbm.at[pl.ds(base, b_per_w)])

    return sc_gather(table_eff, seg_pad)


def kernel(x, W, b, table, seg):
    B, P, DIN = x.shape
    EMB = W.shape[1]

    info = plsc.get_sparse_core_info()
    nc, ns = info.num_cores, info.num_subcores
    nw = nc * ns
    align = 8 * nw
    p_pad = ((P + align - 1) // align) * align

    emb_lanes = 128
    table_eff = jnp.zeros((table.shape[0], emb_lanes), jnp.float32)
    table_eff = table_eff.at[:, :EMB].set(table + b[None, :])
    seg_pad = jnp.concatenate(
        [seg.astype(jnp.int32), jnp.zeros((p_pad - P,), jnp.int32)]
    )

    emb_pad = _sc_gather_rows(table_eff, seg_pad, p_pad, emb_lanes, nw, nc)

    xt = jnp.transpose(x, (1, 2, 0))

    PP = 16
    KK = 8
    wd = jnp.kron(jnp.eye(KK, dtype=W.dtype), W.T)

    def tc_body(xt_ref, wd_ref, emb_ref, out_ref):
        e = emb_ref[:, :EMB][:, :, None]
        for j in range(PP // KK):
            rhs = xt_ref[j * KK:(j + 1) * KK].reshape(KK * DIN, B)
            y = jnp.dot(wd_ref[...], rhs, preferred_element_type=jnp.float32)
            out_ref[j * KK:(j + 1) * KK] = (
                y.reshape(KK, EMB, B) + e[j * KK:(j + 1) * KK]
            )

    out_t = pl.pallas_call(
        tc_body,
        grid=(P // PP,),
        in_specs=[
            pl.BlockSpec((PP, DIN, B), lambda i: (i, 0, 0)),
            pl.BlockSpec((KK * EMB, KK * DIN), lambda i: (0, 0)),
            pl.BlockSpec((PP, emb_lanes), lambda i: (i, 0)),
        ],
        out_specs=pl.BlockSpec((PP, EMB, B), lambda i: (i, 0, 0)),
        out_shape=jax.ShapeDtypeStruct((P, EMB, B), jnp.float32),
    )(xt, wd, emb_pad)

    return jnp.transpose(out_t, (2, 0, 1))


# speedup vs baseline: 6.6787x; 1.0442x over previous
import functools

import jax
import jax.numpy as jnp
from jax.experimental import pallas as pl
from jax.experimental.pallas import tpu as pltpu
from jax.experimental.pallas import tpu_sc as plsc


def _sc_gather_rows(table_eff, seg_pad, n_rows_pad, row_lanes, n_workers, nc):
    b_per_w = n_rows_pad // n_workers

    @functools.partial(
        pl.kernel,
        mesh=plsc.VectorSubcoreMesh(
            core_axis_name="c", subcore_axis_name="s", num_cores=1
        ),
        out_type=jax.ShapeDtypeStruct((n_rows_pad, row_lanes), jnp.float32),
        scratch_types=[
            pltpu.VMEM((b_per_w,), jnp.int32),
            pltpu.VMEM((b_per_w, row_lanes), jnp.float32),
            pltpu.SemaphoreType.DMA,
        ],
    )
    def sc_gather(table_hbm, idx_hbm, out_hbm, idx_v, rows_v, sem):
        wid = jax.lax.axis_index("s") * nc + jax.lax.axis_index("c")
        base = wid * b_per_w
        pltpu.sync_copy(idx_hbm.at[pl.ds(base, b_per_w)], idx_v)
        pltpu.async_copy(table_hbm.at[idx_v], rows_v, sem).wait()
        pltpu.sync_copy(rows_v, out_hbm.at[pl.ds(base, b_per_w)])

    return sc_gather(table_eff, seg_pad)


def kernel(x, W, b, table, seg):
    B, P, DIN = x.shape
    EMB = W.shape[1]

    info = plsc.get_sparse_core_info()
    nc, ns = 1, info.num_subcores
    nw = nc * ns
    align = 8 * nw
    p_pad = ((P + align - 1) // align) * align

    emb_lanes = 128
    table_eff = jnp.zeros((table.shape[0], emb_lanes), jnp.float32)
    table_eff = table_eff.at[:, :EMB].set(table + b[None, :])
    seg_pad = jnp.concatenate(
        [seg.astype(jnp.int32), jnp.zeros((p_pad - P,), jnp.int32)]
    )

    emb_pad = _sc_gather_rows(table_eff, seg_pad, p_pad, emb_lanes, nw, nc)

    xt = jnp.transpose(x, (1, 2, 0))

    PP = 32
    KK = 8
    wd = jnp.kron(jnp.eye(KK, dtype=W.dtype), W.T)

    def tc_body(xt_ref, wd_ref, emb_ref, out_ref):
        e = emb_ref[:, :EMB][:, :, None]
        for j in range(PP // KK):
            rhs = xt_ref[j * KK:(j + 1) * KK].reshape(KK * DIN, B)
            y = jnp.dot(wd_ref[...], rhs, preferred_element_type=jnp.float32)
            out_ref[j * KK:(j + 1) * KK] = (
                y.reshape(KK, EMB, B) + e[j * KK:(j + 1) * KK]
            )

    out_t = pl.pallas_call(
        tc_body,
        grid=(P // PP,),
        in_specs=[
            pl.BlockSpec((PP, DIN, B), lambda i: (i, 0, 0)),
            pl.BlockSpec((KK * EMB, KK * DIN), lambda i: (0, 0)),
            pl.BlockSpec((PP, emb_lanes), lambda i: (i, 0)),
        ],
        out_specs=pl.BlockSpec((PP, EMB, B), lambda i: (i, 0, 0)),
        out_shape=jax.ShapeDtypeStruct((P, EMB, B), jnp.float32),
    )(xt, wd, emb_pad)

    return jnp.transpose(out_t, (2, 0, 1))


# speedup vs baseline: 7.9333x; 1.1878x over previous
import functools

import jax
import jax.numpy as jnp
from jax.experimental import pallas as pl
from jax.experimental.pallas import tpu as pltpu
from jax.experimental.pallas import tpu_sc as plsc


def _sc_gather_rows(table_eff, seg_pad, n_rows_pad, row_lanes, n_workers, nc):
    b_per_w = n_rows_pad // n_workers

    @functools.partial(
        pl.kernel,
        mesh=plsc.VectorSubcoreMesh(
            core_axis_name="c", subcore_axis_name="s", num_cores=1
        ),
        out_type=jax.ShapeDtypeStruct((n_rows_pad, row_lanes), jnp.float32),
        scratch_types=[
            pltpu.VMEM((b_per_w,), jnp.int32),
            pltpu.VMEM((b_per_w, row_lanes), jnp.float32),
            pltpu.SemaphoreType.DMA,
        ],
    )
    def sc_gather(table_hbm, idx_hbm, out_hbm, idx_v, rows_v, sem):
        wid = jax.lax.axis_index("s") * nc + jax.lax.axis_index("c")
        base = wid * b_per_w
        pltpu.sync_copy(idx_hbm.at[pl.ds(base, b_per_w)], idx_v)
        pltpu.async_copy(table_hbm.at[idx_v], rows_v, sem).wait()
        pltpu.sync_copy(rows_v, out_hbm.at[pl.ds(base, b_per_w)])

    return sc_gather(table_eff, seg_pad)


def kernel(x, W, b, table, seg):
    B, P, DIN = x.shape
    EMB = W.shape[1]

    info = plsc.get_sparse_core_info()
    nc, ns = 1, info.num_subcores
    nw = nc * ns
    align = 8 * nw
    p_pad = ((P + align - 1) // align) * align

    emb_lanes = 128
    table_eff = jnp.zeros((table.shape[0], emb_lanes), jnp.float32)
    table_eff = table_eff.at[:, :EMB].set(table + b[None, :])
    n_rows = table.shape[0]
    table_rep = jnp.tile(table_eff, (nw, 1))
    b_per_w = p_pad // nw
    rep_off = (jnp.arange(p_pad, dtype=jnp.int32) // b_per_w) * n_rows
    seg_pad = jnp.concatenate(
        [seg.astype(jnp.int32), jnp.zeros((p_pad - P,), jnp.int32)]
    ) + rep_off

    emb_pad = _sc_gather_rows(table_rep, seg_pad, p_pad, emb_lanes, nw, nc)

    xt = jnp.transpose(x, (1, 2, 0))

    PP = 32
    KK = 8
    wd = jnp.kron(jnp.eye(KK, dtype=W.dtype), W.T)

    def tc_body(xt_ref, wd_ref, emb_ref, out_ref):
        e = emb_ref[:, :EMB][:, :, None]
        for j in range(PP // KK):
            rhs = xt_ref[j * KK:(j + 1) * KK].reshape(KK * DIN, B)
            y = jnp.dot(wd_ref[...], rhs, preferred_element_type=jnp.float32)
            out_ref[j * KK:(j + 1) * KK] = (
                y.reshape(KK, EMB, B) + e[j * KK:(j + 1) * KK]
            )

    out_t = pl.pallas_call(
        tc_body,
        grid=(P // PP,),
        in_specs=[
            pl.BlockSpec((PP, DIN, B), lambda i: (i, 0, 0)),
            pl.BlockSpec((KK * EMB, KK * DIN), lambda i: (0, 0)),
            pl.BlockSpec((PP, emb_lanes), lambda i: (i, 0)),
        ],
        out_specs=pl.BlockSpec((PP, EMB, B), lambda i: (i, 0, 0)),
        out_shape=jax.ShapeDtypeStruct((P, EMB, B), jnp.float32),
    )(xt, wd, emb_pad)

    return jnp.transpose(out_t, (2, 0, 1))


# speedup vs baseline: 8.0754x; 1.0179x over previous
import functools

import jax
import jax.numpy as jnp
from jax.experimental import pallas as pl
from jax.experimental.pallas import tpu as pltpu
from jax.experimental.pallas import tpu_sc as plsc


def _sc_gather_rows(table_eff, seg_pad, n_rows_pad, row_lanes, n_workers, nc):
    b_per_w = n_rows_pad // n_workers

    @functools.partial(
        pl.kernel,
        mesh=plsc.VectorSubcoreMesh(
            core_axis_name="c", subcore_axis_name="s", num_cores=1
        ),
        out_type=jax.ShapeDtypeStruct((n_rows_pad, row_lanes), jnp.float32),
        scratch_types=[
            pltpu.VMEM((b_per_w,), jnp.int32),
            pltpu.VMEM((b_per_w, row_lanes), jnp.float32),
            pltpu.SemaphoreType.DMA,
        ],
    )
    def sc_gather(table_hbm, idx_hbm, out_hbm, idx_v, rows_v, sem):
        wid = jax.lax.axis_index("s") * nc + jax.lax.axis_index("c")
        base = wid * b_per_w
        pltpu.sync_copy(idx_hbm.at[pl.ds(base, b_per_w)], idx_v)
        pltpu.async_copy(table_hbm.at[idx_v], rows_v, sem).wait()
        pltpu.sync_copy(rows_v, out_hbm.at[pl.ds(base, b_per_w)])

    return sc_gather(table_eff, seg_pad)


def kernel(x, W, b, table, seg):
    B, P, DIN = x.shape
    EMB = W.shape[1]

    info = plsc.get_sparse_core_info()
    nc, ns = 1, info.num_subcores
    nw = nc * ns
    align = 8 * nw
    p_pad = ((P + align - 1) // align) * align

    emb_lanes = 128
    table_eff = jnp.zeros((table.shape[0], emb_lanes), jnp.float32)
    table_eff = table_eff.at[:, :EMB].set(table + b[None, :])
    n_rows = table.shape[0]
    table_rep = jnp.tile(table_eff, (nw, 1))
    b_per_w = p_pad // nw
    rep_off = (jnp.arange(p_pad, dtype=jnp.int32) // b_per_w) * n_rows
    seg_pad = jnp.concatenate(
        [seg.astype(jnp.int32), jnp.zeros((p_pad - P,), jnp.int32)]
    ) + rep_off

    emb_pad = _sc_gather_rows(table_rep, seg_pad, p_pad, emb_lanes, nw, nc)

    xt = jnp.transpose(x, (1, 2, 0))

    PP = 64
    KK = 8
    wd = jnp.kron(jnp.eye(KK, dtype=W.dtype), W.T)

    def tc_body(xt_ref, wd_ref, emb_ref, out_ref):
        e = emb_ref[:, :EMB][:, :, None]
        for j in range(PP // KK):
            rhs = xt_ref[j * KK:(j + 1) * KK].reshape(KK * DIN, B)
            y = jnp.dot(wd_ref[...], rhs, preferred_element_type=jnp.float32)
            out_ref[j * KK:(j + 1) * KK] = (
                y.reshape(KK, EMB, B) + e[j * KK:(j + 1) * KK]
            )

    out_t = pl.pallas_call(
        tc_body,
        grid=(P // PP,),
        in_specs=[
            pl.BlockSpec((PP, DIN, B), lambda i: (i, 0, 0)),
            pl.BlockSpec((KK * EMB, KK * DIN), lambda i: (0, 0)),
            pl.BlockSpec((PP, emb_lanes), lambda i: (i, 0)),
        ],
        out_specs=pl.BlockSpec((PP, EMB, B), lambda i: (i, 0, 0)),
        out_shape=jax.ShapeDtypeStruct((P, EMB, B), jnp.float32),
    )(xt, wd, emb_pad)

    return jnp.transpose(out_t, (2, 0, 1))
